# Initial kernel scaffold; baseline (speedup 1.0000x reference)
import jax
import jax.numpy as jnp
from jax.experimental import pallas as pl


def kernel(target, context, negative, in_embed, out_embed):
    raise NotImplementedError("write your pallas kernel here")


# speedup vs baseline: 3.9598x; 3.9598x over previous
import functools

import jax
import jax.numpy as jnp
from jax import lax
from jax.experimental import pallas as pl
from jax.experimental.pallas import tpu as pltpu
from jax.experimental.pallas import tpu_sc as plsc

NC = 2
NS = 16
NW = NC * NS
L = 16

SCORE_W = 32


def _sc_scores(B, K, D, CB):
    T = K + 1
    BW = B // NW
    NCHUNK = BW // CB
    CHW = 96
    NJ = (CB * T) // CHW
    assert CB * T == NJ * CHW and CB % L == 0 and BW % CB == 0

    mesh = plsc.VectorSubcoreMesh(core_axis_name="c", subcore_axis_name="s")

    @functools.partial(
        pl.kernel,
        out_type=jax.ShapeDtypeStruct((B, SCORE_W), jnp.float32),
        mesh=mesh,
        compiler_params=pltpu.CompilerParams(
            needs_layout_passes=False, use_tc_tiling_on_sc=False),
        scratch_types=[
            pltpu.VMEM((BW,), jnp.int32),
            pltpu.VMEM((BW * T,), jnp.int32),
            pltpu.VMEM((CB, D), jnp.float32),
            pltpu.VMEM((CB * T, D), jnp.float32),
            pltpu.VMEM((CB, SCORE_W), jnp.float32),
            pltpu.SemaphoreType.DMA,
        ],
    )
    def scores_kernel(tgt_hbm, cn_hbm, in_hbm, out_hbm, scores_hbm,
                      tidx, cnidx, v_buf, u_buf, sc_buf, sem):
        wid = lax.axis_index("c") * NS + lax.axis_index("s")
        lane = lax.iota(jnp.int32, L)
        pltpu.sync_copy(tgt_hbm.at[pl.ds(wid * BW, BW)], tidx)
        pltpu.sync_copy(cn_hbm.at[pl.ds(wid * BW * T, BW * T)], cnidx)

        def chunk_body(c, _):
            base = wid * BW + c * CB
            copies = [pltpu.async_copy(
                in_hbm.at[tidx.at[pl.ds(c * CB, CB)]], v_buf, sem)]
            for j in range(NJ):
                copies.append(
                    pltpu.async_copy(
                        out_hbm.at[cnidx.at[pl.ds(c * CB * T + j * CHW, CHW)]],
                        u_buf.at[pl.ds(j * CHW, CHW)], sem))
            for cp in copies:
                cp.wait()

            for g in range(CB // L):
                rows16 = lane + g * L
                u_base = lane * T + g * L * T

                def d_body(d, accs):
                    col = jnp.full((L,), d, jnp.int32)
                    vv = plsc.load_gather(v_buf, [rows16, col])
                    return tuple(
                        acc + vv * plsc.load_gather(u_buf, [u_base + t, col])
                        for t, acc in enumerate(accs))

                accs = lax.fori_loop(
                    0, D, d_body,
                    tuple(jnp.zeros((L,), jnp.float32) for _ in range(T)))
                for t, acc in enumerate(accs):
                    plsc.store_scatter(
                        sc_buf, [rows16, jnp.full((L,), t, jnp.int32)], acc)

            pltpu.sync_copy(sc_buf, scores_hbm.at[pl.ds(base, CB)])
            return _

        lax.fori_loop(0, NCHUNK, chunk_body, None)

    return scores_kernel


def _loss_body(s_ref, o_ref):
    s = s_ref[...]
    pos = s[:, 0:1]
    neg = -s[:, 1:]
    def logsig(x):
        return jnp.minimum(x, 0.0) - jnp.log1p(jnp.exp(-jnp.abs(x)))
    total = jnp.sum(logsig(pos)) + jnp.sum(logsig(neg))
    o_ref[...] = jnp.full((1, 1), -total / s.shape[0], jnp.float32)


def kernel(target, context, negative, in_embed, out_embed):
    B, = target.shape
    K = negative.shape[1]
    D = in_embed.shape[1]
    T = K + 1
    CHW = 96
    cn = jnp.concatenate([context[:, None], negative], axis=1).reshape(B * T)

    scores = _sc_scores(B, K, D, CB=64)(target, cn, in_embed, out_embed)

    loss = pl.pallas_call(
        _loss_body,
        out_shape=jax.ShapeDtypeStruct((1, 1), jnp.float32),
    )(scores[:, :T])
    return loss[0, 0]
